# Initial kernel scaffold; baseline (speedup 1.0000x reference)
import jax
import jax.numpy as jnp
from jax.experimental import pallas as pl


def kernel(x, edge_index, id_index, edge_weight, kernel, kernel_id, bias):
    raise NotImplementedError("write your pallas kernel here")


# speedup vs baseline: 11.9841x; 11.9841x over previous
import functools

import jax
import jax.numpy as jnp
from jax import lax
from jax.experimental import pallas as pl
from jax.experimental.pallas import tpu as pltpu
from jax.experimental.pallas import tpu_sc as plsc

_NC = 2
_NS = 16
_NW = _NC * _NS
_CH = 128


def _sc_deg_cnt(row3, w3, id_flat, *, n, n_cnt, id_rows, id_cols):
    cpw = row3.shape[1]
    zc = 400
    assert n % zc == 0 and n_cnt % zc == 0 and id_cols % 16 == 0
    assert id_cols % 8 == 0 and n % 8 == 0

    mesh = plsc.VectorSubcoreMesh(core_axis_name="c", subcore_axis_name="s", num_cores=_NC, num_subcores=_NS)

    @functools.partial(
        pl.kernel,
        out_type=(jax.ShapeDtypeStruct((_NC * n,), jnp.float32),
                  jax.ShapeDtypeStruct((_NC * n_cnt,), jnp.float32)),
        mesh=mesh,
        scratch_types=[
            pltpu.VMEM((cpw, _CH), jnp.int32),
            pltpu.VMEM((cpw, _CH), jnp.float32),
            pltpu.VMEM((id_rows, id_cols), jnp.int32),
            pltpu.VMEM((zc,), jnp.float32),
            pltpu.VMEM((n_cnt,), jnp.float32),
            pltpu.VMEM_SHARED((n,), jnp.float32),
            pltpu.VMEM_SHARED((n_cnt,), jnp.float32),
        ],
    )
    def k(row_h, w_h, id_h, deg_out, cnt_out, rowv, wv, idv, zov, dcv, deg_sh, cnt_sh):
        cid = lax.axis_index("c")
        sid = lax.axis_index("s")
        wid = sid * _NC + cid

        pltpu.sync_copy(row_h.at[wid], rowv)
        pltpu.sync_copy(w_h.at[wid], wv)
        for jj in range(id_rows):
            pltpu.sync_copy(
                id_h.at[pl.ds(wid * id_rows * id_cols + jj * id_cols, id_cols)],
                idv.at[jj])

        @pl.loop(0, zc // 16)
        def _(i):
            zov[pl.ds(i * 16, 16)] = jnp.zeros((16,), jnp.float32)

        @pl.loop(sid, n // zc, step=_NS)
        def _(j):
            pltpu.sync_copy(zov, deg_sh.at[pl.ds(j * zc, zc)])

        @pl.loop(sid, n_cnt // zc, step=_NS)
        def _(j):
            pltpu.sync_copy(zov, cnt_sh.at[pl.ds(j * zc, zc)])

        plsc.subcore_barrier()

        @pl.loop(0, cpw)
        def _(j):
            pltpu.sync_copy(wv.at[j], deg_sh.at[rowv.at[j]], add=True)

        @pl.loop(0, id_cols // 16)
        def _(i):
            zov[pl.ds(i * 16, 16)] = jnp.ones((16,), jnp.float32)

        @pl.loop(0, id_rows)
        def _(j):
            pltpu.sync_copy(zov.at[pl.ds(0, id_cols)], cnt_sh.at[idv.at[j]],
                            add=True)

        plsc.subcore_barrier()

        @pl.when(sid == 0)
        def _():
            pltpu.sync_copy(deg_sh, dcv.at[pl.ds(0, n)])
            pltpu.sync_copy(dcv.at[pl.ds(0, n)], deg_out.at[pl.ds(cid * n, n)])

        @pl.when(sid == 1)
        def _():
            pltpu.sync_copy(cnt_sh, dcv)
            pltpu.sync_copy(dcv, cnt_out.at[pl.ds(cid * n_cnt, n_cnt)])

    return k(row3, w3, id_flat)


def _tc_dense(x, kw, ki, dp, cp):
    n, d = x.shape
    bn = 2000
    assert n % bn == 0

    def body(x_ref, kw_ref, ki_ref, dp_ref, cp_ref, g_ref, dis_ref):
        deg = dp_ref[:, 0:1] + dp_ref[:, 1:2] + 1.0
        dis = jnp.where(deg > 0.0, lax.rsqrt(deg), 0.0)
        cnt = cp_ref[:, 0:1] + cp_ref[:, 1:2]
        xv = x_ref[...]
        h = jnp.dot(xv, kw_ref[...], preferred_element_type=jnp.float32)
        hid = jnp.dot(xv, ki_ref[...], preferred_element_type=jnp.float32)
        h = h + cnt * hid
        g_ref[...] = dis * h
        dis_ref[...] = dis

    return pl.pallas_call(
        body,
        grid=(n // bn,),
        in_specs=[
            pl.BlockSpec((bn, d), lambda i: (i, 0)),
            pl.BlockSpec((d, d), lambda i: (0, 0)),
            pl.BlockSpec((d, d), lambda i: (0, 0)),
            pl.BlockSpec((bn, 2), lambda i: (i, 0)),
            pl.BlockSpec((bn, 2), lambda i: (i, 0)),
        ],
        out_specs=[
            pl.BlockSpec((bn, d), lambda i: (i, 0)),
            pl.BlockSpec((bn, 1), lambda i: (i, 0)),
        ],
        out_shape=(jax.ShapeDtypeStruct((n, d), jnp.float32),
                   jax.ShapeDtypeStruct((n, 1), jnp.float32)),
    )(x, kw, ki, dp, cp)


def _sc_scatter(g, row3, col3, w3):
    n, d = g.shape
    cpw = row3.shape[1]
    st = 1000
    nst = n // st
    assert n % st == 0 and nst <= _NS and d == 128

    mesh = plsc.VectorSubcoreMesh(core_axis_name="c", subcore_axis_name="s", num_cores=_NC, num_subcores=_NS)

    @functools.partial(
        pl.kernel,
        out_type=jax.ShapeDtypeStruct((_NC, n, d), jnp.float32),
        mesh=mesh,
        scratch_types=[
            pltpu.VMEM((cpw, _CH), jnp.int32),
            pltpu.VMEM((cpw, _CH), jnp.int32),
            pltpu.VMEM((cpw, _CH), jnp.float32),
            pltpu.VMEM((_CH, 128), jnp.float32),
            pltpu.VMEM_SHARED((n, 128), jnp.float32),
            pltpu.SemaphoreType.DMA,
        ],
    )
    def k(g_h, row_h, col_h, w_h, s_out, rowv, colv, wv, buf, acc, sem):
        cid = lax.axis_index("c")
        sid = lax.axis_index("s")
        wid = sid * _NC + cid

        pltpu.sync_copy(row_h.at[wid], rowv)
        pltpu.sync_copy(col_h.at[wid], colv)
        pltpu.sync_copy(w_h.at[wid], wv)

        @pl.loop(0, _CH)
        def _(i):
            for dd in range(8):
                buf[i, pl.ds(dd * 16, 16)] = jnp.zeros((16,), jnp.float32)

        @pl.when(sid < nst)
        def _():
            @pl.loop(0, st // _CH)
            def _(j):
                pltpu.sync_copy(buf, acc.at[pl.ds(sid * st + j * _CH, _CH)])
            rem = st % _CH
            if rem:
                pltpu.sync_copy(
                    buf.at[pl.ds(0, rem)],
                    acc.at[pl.ds(sid * st + (st // _CH) * _CH, rem)])

        plsc.subcore_barrier()

        @pl.loop(0, cpw)
        def _(j):
            pltpu.async_copy(g_h.at[colv.at[j]], buf, sem).wait()

            @pl.loop(0, _CH // 16)
            def _(t):
                wvec = wv[j, pl.ds(t * 16, 16)]
                for c in range(16):
                    scal = wvec[c]
                    e = t * 16 + c
                    for dd in range(8):
                        sl = pl.ds(dd * 16, 16)
                        buf[e, sl] = buf[e, sl] * scal

            pltpu.sync_copy(buf, acc.at[rowv.at[j]], add=True)

        plsc.subcore_barrier()

        @pl.when(sid < nst)
        def _():
            pltpu.sync_copy(acc.at[pl.ds(sid * st, st)],
                            s_out.at[cid, pl.ds(sid * st, st)])

    return k(g, row3, col3, w3)


def _tc_combine(sp, g, dis, bias):
    n, d = g.shape
    bn = 2000
    assert n % bn == 0

    def body(sp_ref, g_ref, dis_ref, b_ref, out_ref):
        s = sp_ref[0] + sp_ref[1] + g_ref[...]
        out_ref[...] = dis_ref[...] * s + b_ref[0]

    return pl.pallas_call(
        body,
        grid=(n // bn,),
        in_specs=[
            pl.BlockSpec((2, bn, d), lambda i: (0, i, 0)),
            pl.BlockSpec((bn, d), lambda i: (i, 0)),
            pl.BlockSpec((bn, 1), lambda i: (i, 0)),
            pl.BlockSpec((1, d), lambda i: (0, 0)),
        ],
        out_specs=pl.BlockSpec((bn, d), lambda i: (i, 0)),
        out_shape=jax.ShapeDtypeStruct((n, d), jnp.float32),
    )(sp, g, dis, bias)


def kernel(x, edge_index, id_index, edge_weight, kernel, kernel_id, bias):
    n, d = x.shape
    e = edge_index.shape[1]
    nid = id_index.shape[0]

    row = edge_index[0].astype(jnp.int32)
    col = edge_index[1].astype(jnp.int32)
    w = edge_weight.astype(jnp.float32)

    cpw = -(-e // (_NW * _CH))
    cpw = -(-cpw // 8) * 8
    e_pad = _NW * cpw * _CH
    pad = e_pad - e
    row3 = jnp.concatenate([row, jnp.zeros((pad,), jnp.int32)]).reshape(
        _NW, cpw, _CH)
    col3 = jnp.concatenate([col, jnp.zeros((pad,), jnp.int32)]).reshape(
        _NW, cpw, _CH)
    w3 = jnp.concatenate([w, jnp.zeros((pad,), jnp.float32)]).reshape(
        _NW, cpw, _CH)

    id_cols = 80
    id_rows = -(-nid // (_NW * id_cols))
    nid_pad = _NW * id_rows * id_cols
    id_flat = jnp.concatenate(
        [id_index.astype(jnp.int32),
         jnp.full((nid_pad - nid,), n, jnp.int32)])
    n_cnt = n + 400

    deg_parts, cnt_parts = _sc_deg_cnt(row3, w3, id_flat, n=n, n_cnt=n_cnt,
                                       id_rows=id_rows, id_cols=id_cols)

    g, dis = _tc_dense(x, kernel, kernel_id,
                       deg_parts.reshape(_NC, n).T,
                       cnt_parts.reshape(_NC, n_cnt)[:, :n].T)

    s_parts = _sc_scatter(g, row3, col3, w3)

    return _tc_combine(s_parts, g, dis, bias.reshape(1, d))


# speedup vs baseline: 13.6336x; 1.1376x over previous
import functools

import jax
import jax.numpy as jnp
from jax import lax
from jax.experimental import pallas as pl
from jax.experimental.pallas import tpu as pltpu
from jax.experimental.pallas import tpu_sc as plsc

_NC = 2
_NS = 16
_NW = _NC * _NS
_CH = 128


def _sc_deg_cnt(row3, w3, id_flat, *, n, n_cnt, id_rows, id_cols):
    cpw = row3.shape[1]
    zc = 400
    assert n % zc == 0 and n_cnt % zc == 0 and id_cols % 16 == 0
    assert id_cols % 8 == 0 and n % 8 == 0

    mesh = plsc.VectorSubcoreMesh(core_axis_name="c", subcore_axis_name="s", num_cores=_NC, num_subcores=_NS)

    @functools.partial(
        pl.kernel,
        out_type=(jax.ShapeDtypeStruct((_NC * n,), jnp.float32),
                  jax.ShapeDtypeStruct((_NC * n_cnt,), jnp.float32)),
        mesh=mesh,
        scratch_types=[
            pltpu.VMEM((cpw, _CH), jnp.int32),
            pltpu.VMEM((cpw, _CH), jnp.float32),
            pltpu.VMEM((id_rows, id_cols), jnp.int32),
            pltpu.VMEM((zc,), jnp.float32),
            pltpu.VMEM((n_cnt,), jnp.float32),
            pltpu.VMEM_SHARED((n,), jnp.float32),
            pltpu.VMEM_SHARED((n_cnt,), jnp.float32),
        ],
    )
    def k(row_h, w_h, id_h, deg_out, cnt_out, rowv, wv, idv, zov, dcv, deg_sh, cnt_sh):
        cid = lax.axis_index("c")
        sid = lax.axis_index("s")
        wid = sid * _NC + cid

        pltpu.sync_copy(row_h.at[wid], rowv)
        pltpu.sync_copy(w_h.at[wid], wv)
        for jj in range(id_rows):
            pltpu.sync_copy(
                id_h.at[pl.ds(wid * id_rows * id_cols + jj * id_cols, id_cols)],
                idv.at[jj])

        @pl.loop(0, zc // 16)
        def _(i):
            zov[pl.ds(i * 16, 16)] = jnp.zeros((16,), jnp.float32)

        @pl.loop(sid, n // zc, step=_NS)
        def _(j):
            pltpu.sync_copy(zov, deg_sh.at[pl.ds(j * zc, zc)])

        @pl.loop(sid, n_cnt // zc, step=_NS)
        def _(j):
            pltpu.sync_copy(zov, cnt_sh.at[pl.ds(j * zc, zc)])

        plsc.subcore_barrier()

        @pl.loop(0, cpw)
        def _(j):
            pltpu.sync_copy(wv.at[j], deg_sh.at[rowv.at[j]], add=True)

        @pl.loop(0, id_cols // 16)
        def _(i):
            zov[pl.ds(i * 16, 16)] = jnp.ones((16,), jnp.float32)

        @pl.loop(0, id_rows)
        def _(j):
            pltpu.sync_copy(zov.at[pl.ds(0, id_cols)], cnt_sh.at[idv.at[j]],
                            add=True)

        plsc.subcore_barrier()

        @pl.when(sid == 0)
        def _():
            pltpu.sync_copy(deg_sh, dcv.at[pl.ds(0, n)])
            pltpu.sync_copy(dcv.at[pl.ds(0, n)], deg_out.at[pl.ds(cid * n, n)])

        @pl.when(sid == 1)
        def _():
            pltpu.sync_copy(cnt_sh, dcv)
            pltpu.sync_copy(dcv, cnt_out.at[pl.ds(cid * n_cnt, n_cnt)])

    return k(row3, w3, id_flat)


def _tc_dense(x, kw, ki, dp, cp):
    n, d = x.shape
    bn = 2000
    assert n % bn == 0

    def body(x_ref, kw_ref, ki_ref, dp_ref, cp_ref, g_ref, dis_ref):
        deg = dp_ref[:, 0:1] + dp_ref[:, 1:2] + 1.0
        dis = jnp.where(deg > 0.0, lax.rsqrt(deg), 0.0)
        cnt = cp_ref[:, 0:1] + cp_ref[:, 1:2]
        xv = x_ref[...]
        h = jnp.dot(xv, kw_ref[...], preferred_element_type=jnp.float32)
        hid = jnp.dot(xv, ki_ref[...], preferred_element_type=jnp.float32)
        h = h + cnt * hid
        g_ref[...] = dis * h
        dis_ref[...] = dis

    return pl.pallas_call(
        body,
        grid=(n // bn,),
        in_specs=[
            pl.BlockSpec((bn, d), lambda i: (i, 0)),
            pl.BlockSpec((d, d), lambda i: (0, 0)),
            pl.BlockSpec((d, d), lambda i: (0, 0)),
            pl.BlockSpec((bn, 2), lambda i: (i, 0)),
            pl.BlockSpec((bn, 2), lambda i: (i, 0)),
        ],
        out_specs=[
            pl.BlockSpec((bn, d), lambda i: (i, 0)),
            pl.BlockSpec((bn, 1), lambda i: (i, 0)),
        ],
        out_shape=(jax.ShapeDtypeStruct((n, d), jnp.float32),
                   jax.ShapeDtypeStruct((n, 1), jnp.float32)),
    )(x, kw, ki, dp, cp)


def _sc_scatter(g, row3, col3, w3):
    n, d = g.shape
    cpw = row3.shape[1]
    st = 1000
    nst = n // st
    assert n % st == 0 and nst <= _NS and d == 128

    mesh = plsc.VectorSubcoreMesh(core_axis_name="c", subcore_axis_name="s", num_cores=_NC, num_subcores=_NS)

    @functools.partial(
        pl.kernel,
        out_type=jax.ShapeDtypeStruct((_NC, n, d), jnp.float32),
        mesh=mesh,
        scratch_types=[
            pltpu.VMEM((cpw // 2, _CH), jnp.int32),
            pltpu.VMEM((cpw // 2, _CH), jnp.int32),
            pltpu.VMEM((cpw // 2, _CH), jnp.float32),
            pltpu.VMEM((_CH, 128), jnp.float32),
            pltpu.VMEM((_CH, 128), jnp.float32),
            pltpu.VMEM_SHARED((n, 128), jnp.float32),
            pltpu.SemaphoreType.DMA,
            pltpu.SemaphoreType.DMA,
            pltpu.SemaphoreType.DMA,
            pltpu.SemaphoreType.DMA,
        ],
    )
    def k(g_h, row_h, col_h, w_h, s_out, rowv, colv, wv, bufa, bufb, acc,
          gsa, gsb, ssa, ssb):
        cid = lax.axis_index("c")
        sid = lax.axis_index("s")
        wid = sid * _NC + cid

        @pl.loop(0, _CH)
        def _(i):
            for dd in range(8):
                bufa[i, pl.ds(dd * 16, 16)] = jnp.zeros((16,), jnp.float32)

        @pl.when(sid < nst)
        def _():
            @pl.loop(0, st // _CH)
            def _(j):
                pltpu.sync_copy(bufa, acc.at[pl.ds(sid * st + j * _CH, _CH)])
            rem = st % _CH
            if rem:
                pltpu.sync_copy(
                    bufa.at[pl.ds(0, rem)],
                    acc.at[pl.ds(sid * st + (st // _CH) * _CH, rem)])

        plsc.subcore_barrier()

        bufs = (bufa, bufb)
        gsems = (gsa, gsb)
        ssems = (ssa, ssb)
        hcw = cpw // 2
        nb2 = hcw // 2
        assert hcw % 2 == 0

        for h in range(2):
            pltpu.sync_copy(row_h.at[wid, pl.ds(h * hcw, hcw)], rowv)
            pltpu.sync_copy(col_h.at[wid, pl.ds(h * hcw, hcw)], colv)
            pltpu.sync_copy(w_h.at[wid, pl.ds(h * hcw, hcw)], wv)

            pltpu.async_copy(g_h.at[colv.at[0]], bufa, gsa)

            @pl.loop(0, nb2)
            def _(jj):
                for b in range(2):
                    j = jj * 2 + b
                    buf = bufs[b]
                    ob = 1 - b
                    pltpu.make_async_copy(g_h.at[colv.at[j]], buf,
                                          gsems[b]).wait()
                    if b == 0:
                        @pl.when(jj > 0)
                        def _():
                            pltpu.make_async_copy(
                                bufs[ob], acc.at[rowv.at[j - 1]],
                                ssems[ob]).wait()
                        pltpu.async_copy(g_h.at[colv.at[j + 1]], bufs[ob],
                                         gsems[ob])
                    else:
                        pltpu.make_async_copy(
                            bufs[ob], acc.at[rowv.at[j - 1]],
                            ssems[ob]).wait()

                        @pl.when(jj < nb2 - 1)
                        def _():
                            pltpu.async_copy(g_h.at[colv.at[j + 1]],
                                             bufs[ob], gsems[ob])

                    @pl.loop(0, _CH // 16)
                    def _(t):
                        wvec = wv[j, pl.ds(t * 16, 16)]
                        for c in range(16):
                            scal = wvec[c]
                            e = t * 16 + c
                            for dd in range(8):
                                sl = pl.ds(dd * 16, 16)
                                buf[e, sl] = buf[e, sl] * scal

                    pltpu.async_copy(buf, acc.at[rowv.at[j]], ssems[b],
                                     add=True)

            pltpu.make_async_copy(bufb, acc.at[rowv.at[hcw - 1]], ssb).wait()

        plsc.subcore_barrier()

        @pl.when(sid < nst)
        def _():
            pltpu.sync_copy(acc.at[pl.ds(sid * st, st)],
                            s_out.at[cid, pl.ds(sid * st, st)])

    return k(g, row3, col3, w3)


def _tc_combine(sp, g, dis, bias):
    n, d = g.shape
    bn = 2000
    assert n % bn == 0

    def body(sp_ref, g_ref, dis_ref, b_ref, out_ref):
        s = sp_ref[0] + sp_ref[1] + g_ref[...]
        out_ref[...] = dis_ref[...] * s + b_ref[0]

    return pl.pallas_call(
        body,
        grid=(n // bn,),
        in_specs=[
            pl.BlockSpec((2, bn, d), lambda i: (0, i, 0)),
            pl.BlockSpec((bn, d), lambda i: (i, 0)),
            pl.BlockSpec((bn, 1), lambda i: (i, 0)),
            pl.BlockSpec((1, d), lambda i: (0, 0)),
        ],
        out_specs=pl.BlockSpec((bn, d), lambda i: (i, 0)),
        out_shape=jax.ShapeDtypeStruct((n, d), jnp.float32),
    )(sp, g, dis, bias)


def kernel(x, edge_index, id_index, edge_weight, kernel, kernel_id, bias):
    n, d = x.shape
    e = edge_index.shape[1]
    nid = id_index.shape[0]

    row = edge_index[0].astype(jnp.int32)
    col = edge_index[1].astype(jnp.int32)
    w = edge_weight.astype(jnp.float32)

    cpw = -(-e // (_NW * _CH))
    cpw = -(-cpw // 8) * 8
    e_pad = _NW * cpw * _CH
    pad = e_pad - e
    row3 = jnp.concatenate([row, jnp.zeros((pad,), jnp.int32)]).reshape(
        _NW, cpw, _CH)
    col3 = jnp.concatenate([col, jnp.zeros((pad,), jnp.int32)]).reshape(
        _NW, cpw, _CH)
    w3 = jnp.concatenate([w, jnp.zeros((pad,), jnp.float32)]).reshape(
        _NW, cpw, _CH)

    id_cols = 80
    id_rows = -(-nid // (_NW * id_cols))
    nid_pad = _NW * id_rows * id_cols
    id_flat = jnp.concatenate(
        [id_index.astype(jnp.int32),
         jnp.full((nid_pad - nid,), n, jnp.int32)])
    n_cnt = n + 400

    deg_parts, cnt_parts = _sc_deg_cnt(row3, w3, id_flat, n=n, n_cnt=n_cnt,
                                       id_rows=id_rows, id_cols=id_cols)

    g, dis = _tc_dense(x, kernel, kernel_id,
                       deg_parts.reshape(_NC, n).T,
                       cnt_parts.reshape(_NC, n_cnt)[:, :n].T)

    s_parts = _sc_scatter(g, row3, col3, w3)

    return _tc_combine(s_parts, g, dis, bias.reshape(1, d))
